# baseline (speedup 1.0000x reference)
import functools

import jax
import jax.numpy as jnp
from jax import lax
from jax.experimental import pallas as pl
from jax.experimental.pallas import tpu as pltpu
from jax.experimental.pallas import tpu_sc as plsc

N = 50000
A = 32
D = 32
BL = 25600


_sc_mesh = plsc.ScalarSubcoreMesh(axis_name="c", num_cores=1)


@functools.partial(
    pl.kernel,
    out_type=jax.ShapeDtypeStruct((D, A * 128), jnp.float32),
    mesh=_sc_mesh,
    scratch_types=[
        pltpu.SMEM((A,), jnp.int32),
        pltpu.SemaphoreType.DMA,
    ],
)
def _sc_gather_anchors(et_hbm, idx_hbm, out_hbm, idx_s, sem):
    pltpu.sync_copy(idx_hbm, idx_s)
    copies = []
    for j in range(A):
        col = idx_s[j]
        base = pl.multiple_of(
            lax.shift_left(lax.shift_right_logical(col, 7), 7), 128
        )
        copies.append(pltpu.async_copy(
            et_hbm.at[:, pl.ds(base, 128)],
            out_hbm.at[:, pl.ds(j * 128, 128)],
            sem,
        ))
    for c in copies:
        c.wait()


def _tc_body(dt_ref, et_ref, win_ref, ids_ref, w_ref, b_ref, ot_ref, g1t_ref):
    w = w_ref[...]

    @pl.when(pl.program_id(0) == 0)
    def _():
        r = lax.broadcasted_iota(jnp.int32, (A * 128, A), 0)
        a_idx = lax.broadcasted_iota(jnp.int32, (A * 128, A), 1)
        off = jnp.broadcast_to(
            lax.bitwise_and(ids_ref[...], 127), (A * 128, A)
        )
        oh = (r == a_idx * 128 + off).astype(jnp.float32)
        esel_t = jnp.dot(
            win_ref[...], oh, preferred_element_type=jnp.float32,
        )
        g1t_ref[...] = jnp.dot(
            w[:, :D], esel_t, preferred_element_type=jnp.float32,
        ) * (1.0 / A)

    acc = jnp.dot(
        g1t_ref[...], dt_ref[...],
        preferred_element_type=jnp.float32,
        precision=lax.Precision.DEFAULT,
    )
    acc = acc + jnp.dot(
        w[:, D:], et_ref[...],
        preferred_element_type=jnp.float32,
        precision=lax.Precision.DEFAULT,
    )
    ot_ref[...] = acc + b_ref[...]


def kernel(embeds, anchor_set_id, dists_array, W_hidden, b_hidden):
    ids = anchor_set_id.astype(jnp.int32)
    et = embeds.T
    dt = dists_array.T
    wins = _sc_gather_anchors(et, ids)
    ids2d = ids.reshape(1, A)
    b2d = b_hidden.reshape(D, 1)
    ot = pl.pallas_call(
        _tc_body,
        grid=(pl.cdiv(N, BL),),
        in_specs=[
            pl.BlockSpec((A, BL), lambda i: (0, i)),
            pl.BlockSpec((D, BL), lambda i: (0, i)),
            pl.BlockSpec((D, A * 128), lambda i: (0, 0)),
            pl.BlockSpec((1, A), lambda i: (0, 0)),
            pl.BlockSpec((D, 2 * D), lambda i: (0, 0)),
            pl.BlockSpec((D, 1), lambda i: (0, 0)),
        ],
        out_specs=pl.BlockSpec((D, BL), lambda i: (0, i)),
        out_shape=jax.ShapeDtypeStruct((D, N), jnp.float32),
        scratch_shapes=[pltpu.VMEM((D, A), jnp.float32)],
        compiler_params=pltpu.CompilerParams(
            dimension_semantics=("arbitrary",),
        ),
    )(dt, et, wins, ids2d, W_hidden, b2d)
    return ot.T


# speedup vs baseline: 1.5384x; 1.5384x over previous
import functools

import jax
import jax.numpy as jnp
from jax import lax
from jax.experimental import pallas as pl
from jax.experimental.pallas import tpu as pltpu
from jax.experimental.pallas import tpu_sc as plsc

N = 50000
A = 32
D = 32
BL = 25600


_sc_mesh = plsc.ScalarSubcoreMesh(axis_name="c", num_cores=1)


@functools.partial(
    pl.kernel,
    out_type=jax.ShapeDtypeStruct((D, A * 128), jnp.float32),
    mesh=_sc_mesh,
    scratch_types=[
        pltpu.SMEM((A,), jnp.int32),
        pltpu.VMEM_SHARED((A * D, 128), jnp.float32),
        pltpu.SemaphoreType.DMA,
        pltpu.SemaphoreType.DMA,
    ],
)
def _sc_gather_anchors(et_hbm, idx_hbm, out_hbm, idx_s, win_v, sem, sem2):
    pltpu.sync_copy(idx_hbm, idx_s)
    copies = []
    for j in range(A):
        col = idx_s[j]
        base = pl.multiple_of(
            lax.shift_left(lax.shift_right_logical(col, 7), 7), 128
        )
        copies.append(pltpu.async_copy(
            et_hbm.at[:, pl.ds(base, 128)],
            win_v.at[pl.ds(j * D, D), :],
            sem,
        ))
    for c in copies:
        c.wait()
    outs = []
    for j in range(A):
        outs.append(pltpu.async_copy(
            win_v.at[pl.ds(j * D, D), :],
            out_hbm.at[:, pl.ds(j * 128, 128)],
            sem2,
        ))
    for c in outs:
        c.wait()


def _tc_body(dt_ref, et_ref, win_ref, ids_ref, w_ref, b_ref, ot_ref, g1t_ref):
    w = w_ref[...]

    @pl.when(pl.program_id(0) == 0)
    def _():
        r = lax.broadcasted_iota(jnp.int32, (A * 128, A), 0)
        a_idx = lax.broadcasted_iota(jnp.int32, (A * 128, A), 1)
        off = jnp.broadcast_to(
            lax.bitwise_and(ids_ref[...], 127), (A * 128, A)
        )
        oh = (r == a_idx * 128 + off).astype(jnp.float32)
        esel_t = jnp.dot(
            win_ref[...], oh, preferred_element_type=jnp.float32,
        )
        g1t_ref[...] = jnp.dot(
            w[:, :D], esel_t, preferred_element_type=jnp.float32,
        ) * (1.0 / A)

    acc = jnp.dot(
        g1t_ref[...], dt_ref[...],
        preferred_element_type=jnp.float32,
        precision=lax.Precision.DEFAULT,
    )
    acc = acc + jnp.dot(
        w[:, D:], et_ref[...],
        preferred_element_type=jnp.float32,
        precision=lax.Precision.DEFAULT,
    )
    ot_ref[...] = acc + b_ref[...]


def kernel(embeds, anchor_set_id, dists_array, W_hidden, b_hidden):
    ids = anchor_set_id.astype(jnp.int32)
    et = embeds.T
    dt = dists_array.T
    wins = _sc_gather_anchors(et, ids)
    ids2d = ids.reshape(1, A)
    b2d = b_hidden.reshape(D, 1)
    ot = pl.pallas_call(
        _tc_body,
        grid=(pl.cdiv(N, BL),),
        in_specs=[
            pl.BlockSpec((A, BL), lambda i: (0, i)),
            pl.BlockSpec((D, BL), lambda i: (0, i)),
            pl.BlockSpec((D, A * 128), lambda i: (0, 0)),
            pl.BlockSpec((1, A), lambda i: (0, 0)),
            pl.BlockSpec((D, 2 * D), lambda i: (0, 0)),
            pl.BlockSpec((D, 1), lambda i: (0, 0)),
        ],
        out_specs=pl.BlockSpec((D, BL), lambda i: (0, i)),
        out_shape=jax.ShapeDtypeStruct((D, N), jnp.float32),
        scratch_shapes=[pltpu.VMEM((D, A), jnp.float32)],
        compiler_params=pltpu.CompilerParams(
            dimension_semantics=("arbitrary",),
        ),
    )(dt, et, wins, ids2d, W_hidden, b2d)
    return ot.T
